# Initial kernel scaffold; baseline (speedup 1.0000x reference)
import jax
import jax.numpy as jnp
from jax.experimental import pallas as pl


def kernel(text_emb, struct_feat, edge_index, W_in, b_in, W_gcn, b_gcn, gamma, beta):
    raise NotImplementedError("write your pallas kernel here")


# speedup vs baseline: 7.5601x; 7.5601x over previous
import dataclasses
import functools

import jax
import jax.numpy as jnp
from jax import lax
from jax.experimental import pallas as pl
from jax.experimental.pallas import tpu as pltpu
from jax.experimental.pallas import tpu_sc as plsc

N = 10000
D = 128
E = 320000
TEXT_DIM = 384
STRUCT_DIM = 36
TILES = 32
NSUB = 16
W = 128
KWIN = 160
KDEG = 80
EPAD = NSUB * KWIN * W
NPAD = 10112
NHALF = 5000
ACCR = 5008
BN = 1000
GRID = N // BN
SHIFT = 14
MASK = (1 << SHIFT) - 1


def _sc_degree_body(pk_hbm, part_hbm, idx_v, hist_v, sem):
    c = lax.axis_index("c")
    s = lax.axis_index("s")
    wid = c * 16 + s
    cp = pltpu.async_copy(pk_hbm.at[wid], idx_v, sem)

    @pl.loop(0, NPAD, step=16)
    def _(i):
        hist_v[pl.ds(i, 16)] = jnp.zeros((16,), jnp.float32)

    cp.wait()
    ones16 = jnp.ones((16,), jnp.float32)

    @pl.loop(0, KDEG)
    def _(j):
        @pl.loop(0, W, step=16)
        def _(k):
            dst16 = jnp.right_shift(idx_v[j, pl.ds(k, 16)], SHIFT)
            plsc.addupdate_scatter(hist_v, [dst16], ones16)

    pltpu.sync_copy(hist_v, part_hbm.at[pl.ds(wid * NPAD, NPAD)])


def _sc_scatter_body(g_hbm, pk_hbm, out_hbm,
                     src_v, dst_v, buf, acc_sh, sem_g, sem_i):
    c = lax.axis_index("c")
    s = lax.axis_index("s")
    rbase = s * 312
    cp = pltpu.async_copy(pk_hbm.at[s], src_v, sem_i)

    zb = buf.at[0]

    @pl.loop(0, W)
    def _(i):
        @pl.loop(0, D, step=16)
        def _(j):
            zb[i, pl.ds(j, 16)] = jnp.zeros((16,), jnp.float32)

    @pl.loop(0, 3)
    def _(k):
        off = jnp.minimum(k * W, 312 - W)
        pltpu.sync_copy(zb, acc_sh.at[pl.ds(rbase + off, W)])

    @pl.when(s == NSUB - 1)
    def _():
        pltpu.sync_copy(zb, acc_sh.at[pl.ds(NHALF - W, W)])

    cp.wait()

    hoff = c * NHALF

    @pl.loop(0, KWIN)
    def _(j):
        @pl.loop(0, W, step=16)
        def _(k):
            pk16 = src_v[j, pl.ds(k, 16)]
            d = jnp.right_shift(pk16, SHIFT) - hoff
            ok = jnp.logical_and(d >= 0, d < NHALF)
            dst_v[j, pl.ds(k, 16)] = jnp.where(ok, d, NHALF)
            src_v[j, pl.ds(k, 16)] = jnp.bitwise_and(pk16, MASK)

    plsc.subcore_barrier()

    @pl.loop(0, KWIN)
    def _(j):
        pltpu.async_copy(g_hbm.at[src_v.at[j]], buf.at[0], sem_g).wait()
        pltpu.sync_copy(buf.at[0], acc_sh.at[dst_v.at[j]], add=True)

    plsc.subcore_barrier()
    pltpu.sync_copy(acc_sh.at[pl.ds(rbase, 312)],
                    out_hbm.at[c, pl.ds(rbase, 312)])

    @pl.when(s == NSUB - 1)
    def _():
        pltpu.sync_copy(acc_sh.at[pl.ds(15 * 312 + 312, 16)],
                        out_hbm.at[c, pl.ds(15 * 312 + 312, 16)])


@functools.cache
def _sc_kernels():
    mesh = plsc.VectorSubcoreMesh(core_axis_name="c", subcore_axis_name="s",
                                  num_cores=2, num_subcores=16)
    cp = pltpu.CompilerParams()
    if "needs_layout_passes" in pltpu.CompilerParams.__dataclass_fields__:
        cp = dataclasses.replace(cp, needs_layout_passes=False)
    sc_degree = pl.kernel(
        _sc_degree_body,
        out_type=jax.ShapeDtypeStruct((TILES * NPAD,), jnp.float32),
        mesh=mesh,
        compiler_params=cp,
        scratch_types=[
            pltpu.VMEM((KDEG, W), jnp.int32),
            pltpu.VMEM((NPAD,), jnp.float32),
            pltpu.SemaphoreType.DMA,
        ],
    )
    sc_scatter = pl.kernel(
        _sc_scatter_body,
        out_type=jax.ShapeDtypeStruct((2, ACCR, D), jnp.float32),
        mesh=mesh,
        scratch_types=[
            pltpu.VMEM((KWIN, W), jnp.int32),
            pltpu.VMEM((KWIN, W), jnp.int32),
            pltpu.VMEM((2, W, D), jnp.float32),
            pltpu.VMEM_SHARED((ACCR, D), jnp.float32),
            pltpu.SemaphoreType.DMA,
            pltpu.SemaphoreType.DMA,
        ],
    )
    return sc_degree, sc_scatter


def _gelu(a):
    return 0.5 * a * (1.0 + lax.erf(a * (2.0 ** -0.5)))


def _tc_proj_body(text_ref, struct_ref, wt_ref, ws_ref, b_ref, wg_ref,
                  h0_ref, h_ref):
    a = (jnp.dot(text_ref[...], wt_ref[...], preferred_element_type=jnp.float32)
         + jnp.dot(struct_ref[...], ws_ref[...],
                   preferred_element_type=jnp.float32)
         + b_ref[...])
    h0 = _gelu(a)
    h0_ref[...] = h0
    h_ref[...] = jnp.dot(h0, wg_ref[...], preferred_element_type=jnp.float32)


def _tc_deg_body(p_ref, deg_ref):
    deg_ref[...] = jnp.sum(p_ref[...], axis=0)[:, None] + 1.0


def _tc_scale_body(h_ref, deg_ref, g_ref, dis_ref):
    dis = lax.rsqrt(deg_ref[...])
    dis_ref[...] = dis
    g_ref[...] = h_ref[...] * dis


def _tc_final_body(s_ref, g_ref, dis_ref, h0_ref, bg_ref, gam_ref, bet_ref,
                   o_ref):
    t = s_ref[0] + g_ref[...]
    og = dis_ref[...] * t + bg_ref[...]
    y = _gelu(og) + h0_ref[...]
    mu = jnp.mean(y, axis=-1, keepdims=True)
    dev = y - mu
    var = jnp.mean(dev * dev, axis=-1, keepdims=True)
    o_ref[...] = dev * lax.rsqrt(var + 1e-5) * gam_ref[...] + bet_ref[...]


_tc_proj = pl.pallas_call(
    _tc_proj_body,
    grid=(GRID,),
    in_specs=[
        pl.BlockSpec((BN, TEXT_DIM), lambda i: (i, 0)),
        pl.BlockSpec((BN, STRUCT_DIM), lambda i: (i, 0)),
        pl.BlockSpec((TEXT_DIM, D), lambda i: (0, 0)),
        pl.BlockSpec((STRUCT_DIM, D), lambda i: (0, 0)),
        pl.BlockSpec((1, D), lambda i: (0, 0)),
        pl.BlockSpec((D, D), lambda i: (0, 0)),
    ],
    out_specs=[
        pl.BlockSpec((BN, D), lambda i: (i, 0)),
        pl.BlockSpec((BN, D), lambda i: (i, 0)),
    ],
    out_shape=[
        jax.ShapeDtypeStruct((N, D), jnp.float32),
        jax.ShapeDtypeStruct((N, D), jnp.float32),
    ],
)

_tc_deg = pl.pallas_call(
    _tc_deg_body,
    in_specs=[pl.BlockSpec((TILES, NPAD), lambda: (0, 0))],
    out_specs=pl.BlockSpec((NPAD, 1), lambda: (0, 0)),
    out_shape=jax.ShapeDtypeStruct((NPAD, 1), jnp.float32),
)

_tc_scale = pl.pallas_call(
    _tc_scale_body,
    grid=(GRID,),
    in_specs=[
        pl.BlockSpec((BN, D), lambda i: (i, 0)),
        pl.BlockSpec((BN, 1), lambda i: (i, 0)),
    ],
    out_specs=[
        pl.BlockSpec((BN, D), lambda i: (i, 0)),
        pl.BlockSpec((BN, 1), lambda i: (i, 0)),
    ],
    out_shape=[
        jax.ShapeDtypeStruct((N, D), jnp.float32),
        jax.ShapeDtypeStruct((N, 1), jnp.float32),
    ],
)

_tc_final = pl.pallas_call(
    _tc_final_body,
    grid=(GRID,),
    in_specs=[
        pl.BlockSpec((1, BN, D), lambda i: (i // 5, i % 5, 0)),
        pl.BlockSpec((BN, D), lambda i: (i, 0)),
        pl.BlockSpec((BN, 1), lambda i: (i, 0)),
        pl.BlockSpec((BN, D), lambda i: (i, 0)),
        pl.BlockSpec((1, D), lambda i: (0, 0)),
        pl.BlockSpec((1, D), lambda i: (0, 0)),
        pl.BlockSpec((1, D), lambda i: (0, 0)),
    ],
    out_specs=pl.BlockSpec((BN, D), lambda i: (i, 0)),
    out_shape=jax.ShapeDtypeStruct((N, D), jnp.float32),
)


def kernel(text_emb, struct_feat, edge_index, W_in, b_in, W_gcn, b_gcn,
           gamma, beta):
    sc_degree, sc_scatter = _sc_kernels()
    src = edge_index[0].astype(jnp.int32)
    dst = edge_index[1].astype(jnp.int32)
    packed = jnp.bitwise_or(src, jnp.left_shift(dst, SHIFT))
    pk3 = jnp.concatenate(
        [packed, jnp.full((EPAD - E,), N << SHIFT, jnp.int32)]
    ).reshape(NSUB, KWIN, W)

    partials = sc_degree(pk3.reshape(TILES, KDEG, W))
    h0, h = _tc_proj(text_emb, struct_feat,
                     W_in[:, :TEXT_DIM].T, W_in[:, TEXT_DIM:].T,
                     b_in.reshape(1, D), W_gcn.T)
    deg = _tc_deg(partials.reshape(TILES, NPAD))
    g, dis = _tc_scale(h, deg)
    s_partial = sc_scatter(g, pk3)
    return _tc_final(s_partial, g, dis, h0, b_gcn.reshape(1, D),
                     gamma.reshape(1, D), beta.reshape(1, D))


# speedup vs baseline: 7.6936x; 1.0176x over previous
import dataclasses
import functools

import jax
import jax.numpy as jnp
from jax import lax
from jax.experimental import pallas as pl
from jax.experimental.pallas import tpu as pltpu
from jax.experimental.pallas import tpu_sc as plsc

N = 10000
D = 128
E = 320000
TEXT_DIM = 384
STRUCT_DIM = 36
TILES = 32
NSUB = 16
W = 128
KWIN = 160
KDEG = 80
EPAD = NSUB * KWIN * W
NPAD = 10112
NHALF = 5000
ACCR = 5008
BN = 1000
GRID = N // BN
SHIFT = 14
MASK = (1 << SHIFT) - 1


def _sc_degree_body(pk_hbm, part_hbm, idx_v, hist_v, sem):
    c = lax.axis_index("c")
    s = lax.axis_index("s")
    wid = c * 16 + s
    cp = pltpu.async_copy(pk_hbm.at[wid], idx_v, sem)

    @pl.loop(0, NPAD, step=16)
    def _(i):
        hist_v[pl.ds(i, 16)] = jnp.zeros((16,), jnp.float32)

    cp.wait()
    ones16 = jnp.ones((16,), jnp.float32)

    @pl.loop(0, KDEG)
    def _(j):
        @pl.loop(0, W, step=16)
        def _(k):
            dst16 = jnp.right_shift(idx_v[j, pl.ds(k, 16)], SHIFT)
            plsc.addupdate_scatter(hist_v, [dst16], ones16)

    pltpu.sync_copy(hist_v, part_hbm.at[pl.ds(wid * NPAD, NPAD)])


def _sc_scatter_body(g_hbm, pk_hbm, out_hbm,
                     src_v, dst_v, buf, acc_sh, sem_g, sem_i):
    c = lax.axis_index("c")
    s = lax.axis_index("s")
    rbase = s * 312
    cp = pltpu.async_copy(pk_hbm.at[s], src_v, sem_i)

    zb = buf.at[0]

    @pl.loop(0, W)
    def _(i):
        @pl.loop(0, D, step=16)
        def _(j):
            zb[i, pl.ds(j, 16)] = jnp.zeros((16,), jnp.float32)

    @pl.loop(0, 3)
    def _(k):
        off = jnp.minimum(k * W, 312 - W)
        pltpu.sync_copy(zb, acc_sh.at[pl.ds(rbase + off, W)])

    @pl.when(s == NSUB - 1)
    def _():
        pltpu.sync_copy(zb, acc_sh.at[pl.ds(NHALF - W, W)])

    cp.wait()

    hoff = c * NHALF

    @pl.loop(0, KWIN)
    def _(j):
        @pl.loop(0, W, step=16)
        def _(k):
            pk16 = src_v[j, pl.ds(k, 16)]
            d = jnp.right_shift(pk16, SHIFT) - hoff
            ok = jnp.logical_and(d >= 0, d < NHALF)
            dst_v[j, pl.ds(k, 16)] = jnp.where(ok, d, NHALF)
            src_v[j, pl.ds(k, 16)] = jnp.bitwise_and(pk16, MASK)

    plsc.subcore_barrier()

    @pl.loop(0, KWIN, step=2)
    def _(j):
        cpa = pltpu.async_copy(g_hbm.at[src_v.at[j]], buf.at[0], sem_g)
        cpb = pltpu.async_copy(g_hbm.at[src_v.at[j + 1]], buf.at[1], sem_i)
        cpa.wait()
        pltpu.sync_copy(buf.at[0], acc_sh.at[dst_v.at[j]], add=True)
        cpb.wait()
        pltpu.sync_copy(buf.at[1], acc_sh.at[dst_v.at[j + 1]], add=True)

    plsc.subcore_barrier()
    pltpu.sync_copy(acc_sh.at[pl.ds(rbase, 312)],
                    out_hbm.at[c, pl.ds(rbase, 312)])

    @pl.when(s == NSUB - 1)
    def _():
        pltpu.sync_copy(acc_sh.at[pl.ds(15 * 312 + 312, 16)],
                        out_hbm.at[c, pl.ds(15 * 312 + 312, 16)])


@functools.cache
def _sc_kernels():
    mesh = plsc.VectorSubcoreMesh(core_axis_name="c", subcore_axis_name="s",
                                  num_cores=2, num_subcores=16)
    cp = pltpu.CompilerParams()
    if "needs_layout_passes" in pltpu.CompilerParams.__dataclass_fields__:
        cp = dataclasses.replace(cp, needs_layout_passes=False)
    sc_degree = pl.kernel(
        _sc_degree_body,
        out_type=jax.ShapeDtypeStruct((TILES * NPAD,), jnp.float32),
        mesh=mesh,
        compiler_params=cp,
        scratch_types=[
            pltpu.VMEM((KDEG, W), jnp.int32),
            pltpu.VMEM((NPAD,), jnp.float32),
            pltpu.SemaphoreType.DMA,
        ],
    )
    sc_scatter = pl.kernel(
        _sc_scatter_body,
        out_type=jax.ShapeDtypeStruct((2, ACCR, D), jnp.float32),
        mesh=mesh,
        scratch_types=[
            pltpu.VMEM((KWIN, W), jnp.int32),
            pltpu.VMEM((KWIN, W), jnp.int32),
            pltpu.VMEM((2, W, D), jnp.float32),
            pltpu.VMEM_SHARED((ACCR, D), jnp.float32),
            pltpu.SemaphoreType.DMA,
            pltpu.SemaphoreType.DMA,
        ],
    )
    return sc_degree, sc_scatter


def _gelu(a):
    return 0.5 * a * (1.0 + lax.erf(a * (2.0 ** -0.5)))


def _tc_proj_body(text_ref, struct_ref, wt_ref, ws_ref, b_ref, wg_ref,
                  h0_ref, h_ref):
    a = (jnp.dot(text_ref[...], wt_ref[...], preferred_element_type=jnp.float32)
         + jnp.dot(struct_ref[...], ws_ref[...],
                   preferred_element_type=jnp.float32)
         + b_ref[...])
    h0 = _gelu(a)
    h0_ref[...] = h0
    h_ref[...] = jnp.dot(h0, wg_ref[...], preferred_element_type=jnp.float32)


def _tc_deg_body(p_ref, deg_ref):
    deg_ref[...] = jnp.sum(p_ref[...], axis=0)[:, None] + 1.0


def _tc_scale_body(h_ref, deg_ref, g_ref, dis_ref):
    dis = lax.rsqrt(deg_ref[...])
    dis_ref[...] = dis
    g_ref[...] = h_ref[...] * dis


def _tc_final_body(s_ref, g_ref, dis_ref, h0_ref, bg_ref, gam_ref, bet_ref,
                   o_ref):
    t = s_ref[0] + g_ref[...]
    og = dis_ref[...] * t + bg_ref[...]
    y = _gelu(og) + h0_ref[...]
    mu = jnp.mean(y, axis=-1, keepdims=True)
    dev = y - mu
    var = jnp.mean(dev * dev, axis=-1, keepdims=True)
    o_ref[...] = dev * lax.rsqrt(var + 1e-5) * gam_ref[...] + bet_ref[...]


_tc_proj = pl.pallas_call(
    _tc_proj_body,
    grid=(GRID,),
    in_specs=[
        pl.BlockSpec((BN, TEXT_DIM), lambda i: (i, 0)),
        pl.BlockSpec((BN, STRUCT_DIM), lambda i: (i, 0)),
        pl.BlockSpec((TEXT_DIM, D), lambda i: (0, 0)),
        pl.BlockSpec((STRUCT_DIM, D), lambda i: (0, 0)),
        pl.BlockSpec((1, D), lambda i: (0, 0)),
        pl.BlockSpec((D, D), lambda i: (0, 0)),
    ],
    out_specs=[
        pl.BlockSpec((BN, D), lambda i: (i, 0)),
        pl.BlockSpec((BN, D), lambda i: (i, 0)),
    ],
    out_shape=[
        jax.ShapeDtypeStruct((N, D), jnp.float32),
        jax.ShapeDtypeStruct((N, D), jnp.float32),
    ],
)

_tc_deg = pl.pallas_call(
    _tc_deg_body,
    in_specs=[pl.BlockSpec((TILES, NPAD), lambda: (0, 0))],
    out_specs=pl.BlockSpec((NPAD, 1), lambda: (0, 0)),
    out_shape=jax.ShapeDtypeStruct((NPAD, 1), jnp.float32),
)

_tc_scale = pl.pallas_call(
    _tc_scale_body,
    grid=(GRID,),
    in_specs=[
        pl.BlockSpec((BN, D), lambda i: (i, 0)),
        pl.BlockSpec((BN, 1), lambda i: (i, 0)),
    ],
    out_specs=[
        pl.BlockSpec((BN, D), lambda i: (i, 0)),
        pl.BlockSpec((BN, 1), lambda i: (i, 0)),
    ],
    out_shape=[
        jax.ShapeDtypeStruct((N, D), jnp.float32),
        jax.ShapeDtypeStruct((N, 1), jnp.float32),
    ],
)

_tc_final = pl.pallas_call(
    _tc_final_body,
    grid=(GRID,),
    in_specs=[
        pl.BlockSpec((1, BN, D), lambda i: (i // 5, i % 5, 0)),
        pl.BlockSpec((BN, D), lambda i: (i, 0)),
        pl.BlockSpec((BN, 1), lambda i: (i, 0)),
        pl.BlockSpec((BN, D), lambda i: (i, 0)),
        pl.BlockSpec((1, D), lambda i: (0, 0)),
        pl.BlockSpec((1, D), lambda i: (0, 0)),
        pl.BlockSpec((1, D), lambda i: (0, 0)),
    ],
    out_specs=pl.BlockSpec((BN, D), lambda i: (i, 0)),
    out_shape=jax.ShapeDtypeStruct((N, D), jnp.float32),
)


def kernel(text_emb, struct_feat, edge_index, W_in, b_in, W_gcn, b_gcn,
           gamma, beta):
    sc_degree, sc_scatter = _sc_kernels()
    src = edge_index[0].astype(jnp.int32)
    dst = edge_index[1].astype(jnp.int32)
    packed = jnp.bitwise_or(src, jnp.left_shift(dst, SHIFT))
    pk3 = jnp.concatenate(
        [packed, jnp.full((EPAD - E,), N << SHIFT, jnp.int32)]
    ).reshape(NSUB, KWIN, W)

    partials = sc_degree(pk3.reshape(TILES, KDEG, W))
    h0, h = _tc_proj(text_emb, struct_feat,
                     W_in[:, :TEXT_DIM].T, W_in[:, TEXT_DIM:].T,
                     b_in.reshape(1, D), W_gcn.T)
    deg = _tc_deg(partials.reshape(TILES, NPAD))
    g, dis = _tc_scale(h, deg)
    s_partial = sc_scatter(g, pk3)
    return _tc_final(s_partial, g, dis, h0, b_gcn.reshape(1, D),
                     gamma.reshape(1, D), beta.reshape(1, D))


# speedup vs baseline: 7.8495x; 1.0203x over previous
import dataclasses
import functools

import jax
import jax.numpy as jnp
from jax import lax
from jax.experimental import pallas as pl
from jax.experimental.pallas import tpu as pltpu
from jax.experimental.pallas import tpu_sc as plsc

N = 10000
D = 128
E = 320000
TEXT_DIM = 384
STRUCT_DIM = 36
TILES = 32
NSUB = 16
W = 128
KWIN = 160
KDEG = 80
EPAD = NSUB * KWIN * W
NPAD = 10112
NHALF = 5000
ACCR = 5120
BN = 1000
GRID = N // BN
SHIFT = 14
MASK = (1 << SHIFT) - 1


def _sc_degree_body(pk_hbm, part_hbm, idx_v, hist_v, sem):
    c = lax.axis_index("c")
    s = lax.axis_index("s")
    wid = c * 16 + s
    cp = pltpu.async_copy(pk_hbm.at[wid], idx_v, sem)

    @pl.loop(0, NPAD, step=16)
    def _(i):
        hist_v[pl.ds(i, 16)] = jnp.zeros((16,), jnp.float32)

    cp.wait()
    ones16 = jnp.ones((16,), jnp.float32)

    @pl.loop(0, KDEG)
    def _(j):
        @pl.loop(0, W, step=16)
        def _(k):
            dst16 = jnp.right_shift(idx_v[j, pl.ds(k, 16)], SHIFT)
            plsc.addupdate_scatter(hist_v, [dst16], ones16)

    pltpu.sync_copy(hist_v, part_hbm.at[pl.ds(wid * NPAD, NPAD)])


def _sc_scatter_body(g_hbm, pk_hbm, out_hbm,
                     src_v, dst_v, buf, acc_sh, sem_g, sem_i):
    c = lax.axis_index("c")
    s = lax.axis_index("s")
    rbase = s * 320
    cp = pltpu.async_copy(pk_hbm.at[s], src_v, sem_i)

    zb = buf.at[0]

    @pl.loop(0, W)
    def _(i):
        @pl.loop(0, D, step=16)
        def _(j):
            zb[i, pl.ds(j, 16)] = jnp.zeros((16,), jnp.float32)

    @pl.loop(0, 3)
    def _(k):
        off = jnp.minimum(k * W, 320 - W)
        pltpu.sync_copy(zb, acc_sh.at[pl.ds(rbase + off, W)])

    cp.wait()

    hoff = c * NHALF
    trash = NHALF + s

    @pl.loop(0, KWIN)
    def _(j):
        @pl.loop(0, W, step=16)
        def _(k):
            pk16 = src_v[j, pl.ds(k, 16)]
            d = jnp.right_shift(pk16, SHIFT) - hoff
            ok = jnp.logical_and(d >= 0, d < NHALF)
            dst_v[j, pl.ds(k, 16)] = jnp.where(ok, d, trash)
            src_v[j, pl.ds(k, 16)] = jnp.bitwise_and(pk16, MASK)

    plsc.subcore_barrier()

    @pl.loop(0, KWIN, step=2)
    def _(j):
        cpa = pltpu.async_copy(g_hbm.at[src_v.at[j]], buf.at[0], sem_g)
        cpb = pltpu.async_copy(g_hbm.at[src_v.at[j + 1]], buf.at[1], sem_i)
        cpa.wait()
        pltpu.sync_copy(buf.at[0], acc_sh.at[dst_v.at[j]], add=True)
        cpb.wait()
        pltpu.sync_copy(buf.at[1], acc_sh.at[dst_v.at[j + 1]], add=True)

    plsc.subcore_barrier()
    pltpu.sync_copy(acc_sh.at[pl.ds(rbase, 320)],
                    out_hbm.at[c, pl.ds(rbase, 320)])


@functools.cache
def _sc_kernels():
    mesh = plsc.VectorSubcoreMesh(core_axis_name="c", subcore_axis_name="s",
                                  num_cores=2, num_subcores=16)
    cp = pltpu.CompilerParams()
    if "needs_layout_passes" in pltpu.CompilerParams.__dataclass_fields__:
        cp = dataclasses.replace(cp, needs_layout_passes=False)
    sc_degree = pl.kernel(
        _sc_degree_body,
        out_type=jax.ShapeDtypeStruct((TILES * NPAD,), jnp.float32),
        mesh=mesh,
        compiler_params=cp,
        scratch_types=[
            pltpu.VMEM((KDEG, W), jnp.int32),
            pltpu.VMEM((NPAD,), jnp.float32),
            pltpu.SemaphoreType.DMA,
        ],
    )
    sc_scatter = pl.kernel(
        _sc_scatter_body,
        out_type=jax.ShapeDtypeStruct((2, ACCR, D), jnp.float32),
        mesh=mesh,
        scratch_types=[
            pltpu.VMEM((KWIN, W), jnp.int32),
            pltpu.VMEM((KWIN, W), jnp.int32),
            pltpu.VMEM((2, W, D), jnp.float32),
            pltpu.VMEM_SHARED((ACCR, D), jnp.float32),
            pltpu.SemaphoreType.DMA,
            pltpu.SemaphoreType.DMA,
        ],
    )
    return sc_degree, sc_scatter


def _gelu(a):
    return 0.5 * a * (1.0 + lax.erf(a * (2.0 ** -0.5)))


def _tc_proj_body(text_ref, struct_ref, wt_ref, ws_ref, b_ref, wg_ref,
                  h0_ref, h_ref):
    a = (jnp.dot(text_ref[...], wt_ref[...], preferred_element_type=jnp.float32)
         + jnp.dot(struct_ref[...], ws_ref[...],
                   preferred_element_type=jnp.float32)
         + b_ref[...])
    h0 = _gelu(a)
    h0_ref[...] = h0
    h_ref[...] = jnp.dot(h0, wg_ref[...], preferred_element_type=jnp.float32)


def _tc_deg_body(p_ref, deg_ref):
    deg_ref[...] = jnp.sum(p_ref[...], axis=0)[:, None] + 1.0


def _tc_scale_body(h_ref, deg_ref, g_ref, dis_ref):
    dis = lax.rsqrt(deg_ref[...])
    dis_ref[...] = dis
    g_ref[...] = h_ref[...] * dis


def _tc_final_body(s_ref, g_ref, dis_ref, h0_ref, bg_ref, gam_ref, bet_ref,
                   o_ref):
    t = s_ref[0] + g_ref[...]
    og = dis_ref[...] * t + bg_ref[...]
    y = _gelu(og) + h0_ref[...]
    mu = jnp.mean(y, axis=-1, keepdims=True)
    dev = y - mu
    var = jnp.mean(dev * dev, axis=-1, keepdims=True)
    o_ref[...] = dev * lax.rsqrt(var + 1e-5) * gam_ref[...] + bet_ref[...]


_tc_proj = pl.pallas_call(
    _tc_proj_body,
    grid=(GRID,),
    in_specs=[
        pl.BlockSpec((BN, TEXT_DIM), lambda i: (i, 0)),
        pl.BlockSpec((BN, STRUCT_DIM), lambda i: (i, 0)),
        pl.BlockSpec((TEXT_DIM, D), lambda i: (0, 0)),
        pl.BlockSpec((STRUCT_DIM, D), lambda i: (0, 0)),
        pl.BlockSpec((1, D), lambda i: (0, 0)),
        pl.BlockSpec((D, D), lambda i: (0, 0)),
    ],
    out_specs=[
        pl.BlockSpec((BN, D), lambda i: (i, 0)),
        pl.BlockSpec((BN, D), lambda i: (i, 0)),
    ],
    out_shape=[
        jax.ShapeDtypeStruct((N, D), jnp.float32),
        jax.ShapeDtypeStruct((N, D), jnp.float32),
    ],
)

_tc_deg = pl.pallas_call(
    _tc_deg_body,
    in_specs=[pl.BlockSpec((TILES, NPAD), lambda: (0, 0))],
    out_specs=pl.BlockSpec((NPAD, 1), lambda: (0, 0)),
    out_shape=jax.ShapeDtypeStruct((NPAD, 1), jnp.float32),
)

_tc_scale = pl.pallas_call(
    _tc_scale_body,
    grid=(GRID,),
    in_specs=[
        pl.BlockSpec((BN, D), lambda i: (i, 0)),
        pl.BlockSpec((BN, 1), lambda i: (i, 0)),
    ],
    out_specs=[
        pl.BlockSpec((BN, D), lambda i: (i, 0)),
        pl.BlockSpec((BN, 1), lambda i: (i, 0)),
    ],
    out_shape=[
        jax.ShapeDtypeStruct((N, D), jnp.float32),
        jax.ShapeDtypeStruct((N, 1), jnp.float32),
    ],
)

_tc_final = pl.pallas_call(
    _tc_final_body,
    grid=(GRID,),
    in_specs=[
        pl.BlockSpec((1, BN, D), lambda i: (i // 5, i % 5, 0)),
        pl.BlockSpec((BN, D), lambda i: (i, 0)),
        pl.BlockSpec((BN, 1), lambda i: (i, 0)),
        pl.BlockSpec((BN, D), lambda i: (i, 0)),
        pl.BlockSpec((1, D), lambda i: (0, 0)),
        pl.BlockSpec((1, D), lambda i: (0, 0)),
        pl.BlockSpec((1, D), lambda i: (0, 0)),
    ],
    out_specs=pl.BlockSpec((BN, D), lambda i: (i, 0)),
    out_shape=jax.ShapeDtypeStruct((N, D), jnp.float32),
)


def kernel(text_emb, struct_feat, edge_index, W_in, b_in, W_gcn, b_gcn,
           gamma, beta):
    sc_degree, sc_scatter = _sc_kernels()
    src = edge_index[0].astype(jnp.int32)
    dst = edge_index[1].astype(jnp.int32)
    packed = jnp.bitwise_or(src, jnp.left_shift(dst, SHIFT))
    pk3 = jnp.concatenate(
        [packed, jnp.full((EPAD - E,), N << SHIFT, jnp.int32)]
    ).reshape(NSUB, KWIN, W)

    partials = sc_degree(pk3.reshape(TILES, KDEG, W))
    h0, h = _tc_proj(text_emb, struct_feat,
                     W_in[:, :TEXT_DIM].T, W_in[:, TEXT_DIM:].T,
                     b_in.reshape(1, D), W_gcn.T)
    deg = _tc_deg(partials.reshape(TILES, NPAD))
    g, dis = _tc_scale(h, deg)
    s_partial = sc_scatter(g, pk3)
    return _tc_final(s_partial, g, dis, h0, b_gcn.reshape(1, D),
                     gamma.reshape(1, D), beta.reshape(1, D))


# speedup vs baseline: 20.5331x; 2.6159x over previous
import dataclasses
import functools

import jax
import jax.numpy as jnp
from jax import lax
from jax.experimental import pallas as pl
from jax.experimental.pallas import tpu as pltpu
from jax.experimental.pallas import tpu_sc as plsc

N = 10000
D = 128
E = 320000
TEXT_DIM = 384
STRUCT_DIM = 36
TILES = 32
NSUB = 16
W = 128
KWIN = 160
KDEG = 80
EPAD = NSUB * KWIN * W
EDGT = KWIN * W
NPAD = 10112
NHALF = 5000
ACCR = 5120
BN = 1000
GRID = N // BN
SHIFT = 14
MASK = (1 << SHIFT) - 1


def _sc_degree_body(pk_hbm, part_hbm, idx_v, hist_v, sem):
    c = lax.axis_index("c")
    s = lax.axis_index("s")
    wid = c * 16 + s
    cp = pltpu.async_copy(pk_hbm.at[wid], idx_v, sem)

    @pl.loop(0, NPAD, step=16)
    def _(i):
        hist_v[pl.ds(i, 16)] = jnp.zeros((16,), jnp.float32)

    cp.wait()
    ones16 = jnp.ones((16,), jnp.float32)

    @pl.loop(0, KDEG)
    def _(j):
        @pl.loop(0, W, step=16)
        def _(k):
            dst16 = jnp.right_shift(idx_v[j, pl.ds(k, 16)], SHIFT)
            plsc.addupdate_scatter(hist_v, [dst16], ones16)

    pltpu.sync_copy(hist_v, part_hbm.at[pl.ds(wid * NPAD, NPAD)])


def _sc_scatter_body(g_hbm, pk_hbm, out_hbm,
                     srcf_v, dstf_v, dst2_v, buf, acc_sh, sem_g, sem_i):
    c = lax.axis_index("c")
    s = lax.axis_index("s")
    rbase = s * 320
    cp = pltpu.async_copy(pk_hbm.at[s], dst2_v, sem_i)

    zb = buf.at[0]

    @pl.loop(0, W)
    def _(i):
        @pl.loop(0, D, step=16)
        def _(j):
            zb[i, pl.ds(j, 16)] = jnp.zeros((16,), jnp.float32)

    @pl.loop(0, 3)
    def _(k):
        off = jnp.minimum(k * W, 320 - W)
        pltpu.sync_copy(zb, acc_sh.at[pl.ds(rbase + off, W)])

    hoff = c * NHALF
    trash = NHALF + s
    zeros16 = jnp.zeros((16,), jnp.int32)
    trash16 = jnp.broadcast_to(trash, (16,))

    @pl.loop(0, EDGT, step=16)
    def _(i):
        srcf_v[pl.ds(i, 16)] = zeros16
        dstf_v[pl.ds(i, 16)] = trash16

    cp.wait()

    def _compact(i, n):
        pk16 = dst2_v[i // (W // 16), pl.ds((i % (W // 16)) * 16, 16)]
        d = jnp.right_shift(pk16, SHIFT) - hoff
        ok = jnp.logical_and(d >= 0, d < NHALF)
        plsc.store_compressed(srcf_v.at[pl.ds(n, 16)],
                              jnp.bitwise_and(pk16, MASK), mask=ok)
        plsc.store_compressed(dstf_v.at[pl.ds(n, 16)], d, mask=ok)
        return n + jnp.sum(ok.astype(jnp.int32))

    n = lax.fori_loop(0, EDGT // 16, _compact, 0)
    nw = (n + W - 1) // W

    @pl.loop(0, KWIN)
    def _(r):
        @pl.loop(0, W, step=16)
        def _(k):
            dst2_v[r, pl.ds(k, 16)] = dstf_v[pl.ds(r * W + k, 16)]

    plsc.subcore_barrier()

    def _window(j, carry):
        pltpu.async_copy(g_hbm.at[srcf_v.at[pl.ds(j * W, W)]],
                         buf.at[0], sem_g).wait()
        pltpu.sync_copy(buf.at[0], acc_sh.at[dst2_v.at[j]], add=True)
        return carry

    lax.fori_loop(0, nw, _window, 0)

    plsc.subcore_barrier()
    pltpu.sync_copy(acc_sh.at[pl.ds(rbase, 320)],
                    out_hbm.at[c, pl.ds(rbase, 320)])


@functools.cache
def _sc_kernels():
    mesh = plsc.VectorSubcoreMesh(core_axis_name="c", subcore_axis_name="s",
                                  num_cores=2, num_subcores=16)
    cp = pltpu.CompilerParams()
    if "needs_layout_passes" in pltpu.CompilerParams.__dataclass_fields__:
        cp = dataclasses.replace(cp, needs_layout_passes=False)
    sc_degree = pl.kernel(
        _sc_degree_body,
        out_type=jax.ShapeDtypeStruct((TILES * NPAD,), jnp.float32),
        mesh=mesh,
        compiler_params=cp,
        scratch_types=[
            pltpu.VMEM((KDEG, W), jnp.int32),
            pltpu.VMEM((NPAD,), jnp.float32),
            pltpu.SemaphoreType.DMA,
        ],
    )
    sc_scatter = pl.kernel(
        _sc_scatter_body,
        out_type=jax.ShapeDtypeStruct((2, ACCR, D), jnp.float32),
        mesh=mesh,
        compiler_params=cp,
        scratch_types=[
            pltpu.VMEM((EDGT,), jnp.int32),
            pltpu.VMEM((EDGT,), jnp.int32),
            pltpu.VMEM((KWIN, W), jnp.int32),
            pltpu.VMEM((1, W, D), jnp.float32),
            pltpu.VMEM_SHARED((ACCR, D), jnp.float32),
            pltpu.SemaphoreType.DMA,
            pltpu.SemaphoreType.DMA,
        ],
    )
    return sc_degree, sc_scatter


def _gelu(a):
    return 0.5 * a * (1.0 + lax.erf(a * (2.0 ** -0.5)))


def _tc_proj_body(text_ref, struct_ref, wt_ref, ws_ref, b_ref, wg_ref,
                  h0_ref, h_ref):
    a = (jnp.dot(text_ref[...], wt_ref[...], preferred_element_type=jnp.float32)
         + jnp.dot(struct_ref[...], ws_ref[...],
                   preferred_element_type=jnp.float32)
         + b_ref[...])
    h0 = _gelu(a)
    h0_ref[...] = h0
    h_ref[...] = jnp.dot(h0, wg_ref[...], preferred_element_type=jnp.float32)


def _tc_deg_body(p_ref, deg_ref):
    deg_ref[...] = jnp.sum(p_ref[...], axis=0)[:, None] + 1.0


def _tc_scale_body(h_ref, deg_ref, g_ref, dis_ref):
    dis = lax.rsqrt(deg_ref[...])
    dis_ref[...] = dis
    g_ref[...] = h_ref[...] * dis


def _tc_final_body(s_ref, g_ref, dis_ref, h0_ref, bg_ref, gam_ref, bet_ref,
                   o_ref):
    t = s_ref[0] + g_ref[...]
    og = dis_ref[...] * t + bg_ref[...]
    y = _gelu(og) + h0_ref[...]
    mu = jnp.mean(y, axis=-1, keepdims=True)
    dev = y - mu
    var = jnp.mean(dev * dev, axis=-1, keepdims=True)
    o_ref[...] = dev * lax.rsqrt(var + 1e-5) * gam_ref[...] + bet_ref[...]


_tc_proj = pl.pallas_call(
    _tc_proj_body,
    grid=(GRID,),
    in_specs=[
        pl.BlockSpec((BN, TEXT_DIM), lambda i: (i, 0)),
        pl.BlockSpec((BN, STRUCT_DIM), lambda i: (i, 0)),
        pl.BlockSpec((TEXT_DIM, D), lambda i: (0, 0)),
        pl.BlockSpec((STRUCT_DIM, D), lambda i: (0, 0)),
        pl.BlockSpec((1, D), lambda i: (0, 0)),
        pl.BlockSpec((D, D), lambda i: (0, 0)),
    ],
    out_specs=[
        pl.BlockSpec((BN, D), lambda i: (i, 0)),
        pl.BlockSpec((BN, D), lambda i: (i, 0)),
    ],
    out_shape=[
        jax.ShapeDtypeStruct((N, D), jnp.float32),
        jax.ShapeDtypeStruct((N, D), jnp.float32),
    ],
)

_tc_deg = pl.pallas_call(
    _tc_deg_body,
    in_specs=[pl.BlockSpec((TILES, NPAD), lambda: (0, 0))],
    out_specs=pl.BlockSpec((NPAD, 1), lambda: (0, 0)),
    out_shape=jax.ShapeDtypeStruct((NPAD, 1), jnp.float32),
)

_tc_scale = pl.pallas_call(
    _tc_scale_body,
    grid=(GRID,),
    in_specs=[
        pl.BlockSpec((BN, D), lambda i: (i, 0)),
        pl.BlockSpec((BN, 1), lambda i: (i, 0)),
    ],
    out_specs=[
        pl.BlockSpec((BN, D), lambda i: (i, 0)),
        pl.BlockSpec((BN, 1), lambda i: (i, 0)),
    ],
    out_shape=[
        jax.ShapeDtypeStruct((N, D), jnp.float32),
        jax.ShapeDtypeStruct((N, 1), jnp.float32),
    ],
)

_tc_final = pl.pallas_call(
    _tc_final_body,
    grid=(GRID,),
    in_specs=[
        pl.BlockSpec((1, BN, D), lambda i: (i // 5, i % 5, 0)),
        pl.BlockSpec((BN, D), lambda i: (i, 0)),
        pl.BlockSpec((BN, 1), lambda i: (i, 0)),
        pl.BlockSpec((BN, D), lambda i: (i, 0)),
        pl.BlockSpec((1, D), lambda i: (0, 0)),
        pl.BlockSpec((1, D), lambda i: (0, 0)),
        pl.BlockSpec((1, D), lambda i: (0, 0)),
    ],
    out_specs=pl.BlockSpec((BN, D), lambda i: (i, 0)),
    out_shape=jax.ShapeDtypeStruct((N, D), jnp.float32),
)


def kernel(text_emb, struct_feat, edge_index, W_in, b_in, W_gcn, b_gcn,
           gamma, beta):
    sc_degree, sc_scatter = _sc_kernels()
    src = edge_index[0].astype(jnp.int32)
    dst = edge_index[1].astype(jnp.int32)
    packed = jnp.bitwise_or(src, jnp.left_shift(dst, SHIFT))
    pk3 = jnp.concatenate(
        [packed, jnp.full((EPAD - E,), N << SHIFT, jnp.int32)]
    ).reshape(NSUB, KWIN, W)

    partials = sc_degree(pk3.reshape(TILES, KDEG, W))
    h0, h = _tc_proj(text_emb, struct_feat,
                     W_in[:, :TEXT_DIM].T, W_in[:, TEXT_DIM:].T,
                     b_in.reshape(1, D), W_gcn.T)
    deg = _tc_deg(partials.reshape(TILES, NPAD))
    g, dis = _tc_scale(h, deg)
    s_partial = sc_scatter(g, pk3)
    return _tc_final(s_partial, g, dis, h0, b_gcn.reshape(1, D),
                     gamma.reshape(1, D), beta.reshape(1, D))
